# baseline (speedup 1.0000x reference)
import functools

import jax
import jax.numpy as jnp
from jax import lax
from jax.experimental import pallas as pl
from jax.experimental.pallas import tpu as pltpu
from jax.experimental.pallas import tpu_sc as plsc

B, K, C, NPOINT = 8, 20000, 256, 4096
NW = 32
WPB = NW // B
PPW = NPOINT // WPB
NCH = 128
NHF = NCH // 2
NCHUNK = PPW // NCH
TILES = (NHF // 16) * (C // 16)

_mesh = plsc.VectorSubcoreMesh(core_axis_name="c", subcore_axis_name="s")


@functools.partial(
    pl.kernel,
    mesh=_mesh,
    out_type=(
        jax.ShapeDtypeStruct((3, B, NPOINT), jnp.float32),
        jax.ShapeDtypeStruct((B, C, NPOINT), jnp.float32),
    ),
    compiler_params=pltpu.CompilerParams(needs_layout_passes=False),
    scratch_types=[
        pltpu.VMEM((PPW,), jnp.int32),
        pltpu.VMEM((NHF,), jnp.int32),
        pltpu.VMEM((NHF,), jnp.int32),
        pltpu.VMEM((NHF, C), jnp.float32),
        pltpu.VMEM((NHF, C), jnp.float32),
        pltpu.VMEM((C, NCH), jnp.float32),
        pltpu.VMEM((C, NCH), jnp.float32),
        pltpu.VMEM((NPOINT,), jnp.int32),
        pltpu.VMEM((K,), jnp.float32),
        pltpu.VMEM((NPOINT,), jnp.float32),
        pltpu.SemaphoreType.DMA,
        pltpu.SemaphoreType.DMA,
        pltpu.SemaphoreType.DMA,
        pltpu.SemaphoreType.DMA,
    ],
)
def _sc_gather(xyzp_hbm, featr_hbm, idxq_hbm, idx_hbm, oxyz_hbm, ofeat_hbm,
               idxq_v, gidx0_v, gidx1_v, src0_v, src1_v, dst0_v, dst1_v,
               idxf_v, xrow_v, oxrow_v, sem_i0, sem_i1, sem_o0, sem_o1):
    cid = lax.axis_index("c")
    sid = lax.axis_index("s")
    wid = sid * 2 + cid
    b = wid // WPB
    q = wid % WPB

    iota = lax.iota(jnp.int32, 16)
    gidx = [gidx0_v, gidx1_v]
    src = [src0_v, src1_v]
    dst = [dst0_v, dst1_v]
    sem_in = [sem_i0, sem_i1]
    sem_out = [sem_o0, sem_o1]
    rbase = b * K
    perm = [(iota + d) & 15 for d in range(16)]

    def fill_gidx(j, h, gbuf):
        for u in range(NHF // 16):
            iv = idxq_v[pl.ds(j * NCH + h * NHF + u * 16, 16)]
            gbuf[pl.ds(u * 16, 16)] = iv + rbase

    pltpu.sync_copy(idxq_hbm.at[wid], idxq_v)
    for h in range(2):
        fill_gidx(0, h, gidx[h])
        pltpu.async_copy(featr_hbm.at[gidx[h]], src[h], sem_in[h])

    @pl.when(wid < 24)
    def _():
        xd = wid % 3
        xb = wid // 3
        pltpu.sync_copy(idx_hbm.at[xb], idxf_v)
        pltpu.sync_copy(xyzp_hbm.at[xd, xb], xrow_v)

        @plsc.parallel_loop(0, NPOINT // 16, unroll=4)
        def xyz_body(k):
            iv = idxf_v[pl.ds(k * 16, 16)]
            oxrow_v[pl.ds(k * 16, 16)] = plsc.load_gather(xrow_v, [iv])

        pltpu.sync_copy(oxrow_v, oxyz_hbm.at[xd, xb])

    def transpose_half(sb, db, h):
        @plsc.parallel_loop(0, TILES, unroll=4)
        def tile_body(z):
            tn = z // (C // 16)
            tc = z % (C // 16)
            rows = tn * 16 + iota
            c0 = tc * 16
            for d in range(16):
                cols = c0 + perm[d]
                v = plsc.load_gather(sb, [rows, cols])
                plsc.store_scatter(db, [cols, rows + h * NHF], v)

    def step_body(t, _):
        for i in range(2):
            j = t * 2 + i
            db = dst[i]

            @pl.when(t > 0)
            def _():
                pltpu.make_async_copy(
                    db, ofeat_hbm.at[b, :, pl.ds(0, NCH)], sem_out[i]).wait()

            for h in range(2):
                pltpu.make_async_copy(
                    featr_hbm.at[gidx[h]], src[h], sem_in[h]).wait()
                transpose_half(src[h], db, h)

                @pl.when(j < NCHUNK - 1)
                def _():
                    fill_gidx(j + 1, h, gidx[h])
                    pltpu.async_copy(featr_hbm.at[gidx[h]], src[h], sem_in[h])

            n0 = q * PPW + j * NCH
            pltpu.async_copy(db, ofeat_hbm.at[b, :, pl.ds(n0, NCH)], sem_out[i])

        return 0

    lax.fori_loop(0, NCHUNK // 2, step_body, 0)

    for i in range(2):
        pltpu.make_async_copy(
            dst[i], ofeat_hbm.at[b, :, pl.ds(0, NCH)], sem_out[i]).wait()


def kernel(xyz, features, sample_inds):
    idx32 = sample_inds.astype(jnp.int32)
    idxq = idx32.reshape(NW, PPW)
    xyzp = xyz.transpose(2, 0, 1)
    featr = features.transpose(0, 2, 1).reshape(B * K, C)
    oxyz, ofeat = _sc_gather(xyzp, featr, idxq, idx32)
    new_xyz = oxyz.transpose(1, 2, 0)
    return (new_xyz, ofeat, sample_inds)


# speedup vs baseline: 1.0001x; 1.0001x over previous
import functools

import jax
import jax.numpy as jnp
from jax import lax
from jax.experimental import pallas as pl
from jax.experimental.pallas import tpu as pltpu
from jax.experimental.pallas import tpu_sc as plsc

B, K, C, NPOINT = 8, 20000, 256, 4096
NW = 32
WPB = NW // B
PPW = NPOINT // WPB
NCH = 128
NHF = NCH // 2
NCHUNK = PPW // NCH
TILES = (NHF // 16) * (C // 16)

_mesh = plsc.VectorSubcoreMesh(core_axis_name="c", subcore_axis_name="s")


@functools.partial(
    pl.kernel,
    mesh=_mesh,
    out_type=(
        jax.ShapeDtypeStruct((3, B, NPOINT), jnp.float32),
        jax.ShapeDtypeStruct((B, C, NPOINT), jnp.float32),
    ),
    compiler_params=pltpu.CompilerParams(needs_layout_passes=False, disable_bounds_checks=True),
    scratch_types=[
        pltpu.VMEM((PPW,), jnp.int32),
        pltpu.VMEM((NHF,), jnp.int32),
        pltpu.VMEM((NHF,), jnp.int32),
        pltpu.VMEM((NHF, C), jnp.float32),
        pltpu.VMEM((NHF, C), jnp.float32),
        pltpu.VMEM((C, NCH), jnp.float32),
        pltpu.VMEM((C, NCH), jnp.float32),
        pltpu.VMEM((NPOINT,), jnp.int32),
        pltpu.VMEM((K,), jnp.float32),
        pltpu.VMEM((NPOINT,), jnp.float32),
        pltpu.SemaphoreType.DMA,
        pltpu.SemaphoreType.DMA,
        pltpu.SemaphoreType.DMA,
        pltpu.SemaphoreType.DMA,
    ],
)
def _sc_gather(xyzp_hbm, featr_hbm, idxq_hbm, idx_hbm, oxyz_hbm, ofeat_hbm,
               idxq_v, gidx0_v, gidx1_v, src0_v, src1_v, dst0_v, dst1_v,
               idxf_v, xrow_v, oxrow_v, sem_i0, sem_i1, sem_o0, sem_o1):
    cid = lax.axis_index("c")
    sid = lax.axis_index("s")
    wid = sid * 2 + cid
    b = wid // WPB
    q = wid % WPB

    iota = lax.iota(jnp.int32, 16)
    gidx = [gidx0_v, gidx1_v]
    src = [src0_v, src1_v]
    dst = [dst0_v, dst1_v]
    sem_in = [sem_i0, sem_i1]
    sem_out = [sem_o0, sem_o1]
    rbase = b * K
    perm = [(iota + d) & 15 for d in range(16)]

    def fill_gidx(j, h, gbuf):
        for u in range(NHF // 16):
            iv = idxq_v[pl.ds(j * NCH + h * NHF + u * 16, 16)]
            gbuf[pl.ds(u * 16, 16)] = iv + rbase

    pltpu.sync_copy(idxq_hbm.at[wid], idxq_v)
    for h in range(2):
        fill_gidx(0, h, gidx[h])
        pltpu.async_copy(featr_hbm.at[gidx[h]], src[h], sem_in[h])

    @pl.when(wid < 24)
    def _():
        xd = wid % 3
        xb = wid // 3
        pltpu.sync_copy(idx_hbm.at[xb], idxf_v)
        pltpu.sync_copy(xyzp_hbm.at[xd, xb], xrow_v)

        @plsc.parallel_loop(0, NPOINT // 16, unroll=4)
        def xyz_body(k):
            iv = idxf_v[pl.ds(k * 16, 16)]
            oxrow_v[pl.ds(k * 16, 16)] = plsc.load_gather(xrow_v, [iv])

        pltpu.sync_copy(oxrow_v, oxyz_hbm.at[xd, xb])

    def transpose_half(sb, db, h):
        @plsc.parallel_loop(0, TILES, unroll=2)
        def tile_body(z):
            tn = z // (C // 16)
            tc = z % (C // 16)
            rows = tn * 16 + iota
            c0 = tc * 16
            for d in range(16):
                cols = c0 + perm[d]
                v = plsc.load_gather(sb, [rows, cols])
                plsc.store_scatter(db, [cols, rows + h * NHF], v)

    def step_body(t, _):
        for i in range(2):
            j = t * 2 + i
            db = dst[i]

            @pl.when(t > 0)
            def _():
                pltpu.make_async_copy(
                    db, ofeat_hbm.at[b, :, pl.ds(0, NCH)], sem_out[i]).wait()

            for h in range(2):
                pltpu.make_async_copy(
                    featr_hbm.at[gidx[h]], src[h], sem_in[h]).wait()
                transpose_half(src[h], db, h)

                @pl.when(j < NCHUNK - 1)
                def _():
                    fill_gidx(j + 1, h, gidx[h])
                    pltpu.async_copy(featr_hbm.at[gidx[h]], src[h], sem_in[h])

            n0 = q * PPW + j * NCH
            pltpu.async_copy(db, ofeat_hbm.at[b, :, pl.ds(n0, NCH)], sem_out[i])

        return 0

    lax.fori_loop(0, NCHUNK // 2, step_body, 0)

    for i in range(2):
        pltpu.make_async_copy(
            dst[i], ofeat_hbm.at[b, :, pl.ds(0, NCH)], sem_out[i]).wait()


def kernel(xyz, features, sample_inds):
    idx32 = sample_inds.astype(jnp.int32)
    idxq = idx32.reshape(NW, PPW)
    xyzp = xyz.transpose(2, 0, 1)
    featr = features.transpose(0, 2, 1).reshape(B * K, C)
    oxyz, ofeat = _sc_gather(xyzp, featr, idxq, idx32)
    new_xyz = oxyz.transpose(1, 2, 0)
    return (new_xyz, ofeat, sample_inds)


# speedup vs baseline: 1.0052x; 1.0052x over previous
import functools

import jax
import jax.numpy as jnp
from jax import lax
from jax.experimental import pallas as pl
from jax.experimental.pallas import tpu as pltpu
from jax.experimental.pallas import tpu_sc as plsc

B, K, C, NPOINT = 8, 20000, 256, 4096
NW = 32
WPB = NW // B
PPW = NPOINT // WPB
NCH = 128
NHF = NCH // 2
NCHUNK = PPW // NCH
TILES = (NHF // 16) * (C // 16)

_mesh = plsc.VectorSubcoreMesh(core_axis_name="c", subcore_axis_name="s")


@functools.partial(
    pl.kernel,
    mesh=_mesh,
    out_type=(
        jax.ShapeDtypeStruct((3, B, NPOINT), jnp.float32),
        jax.ShapeDtypeStruct((B, C, NPOINT), jnp.float32),
    ),
    compiler_params=pltpu.CompilerParams(needs_layout_passes=False, disable_bounds_checks=True),
    scratch_types=[
        pltpu.VMEM((PPW,), jnp.int32),
        pltpu.VMEM((NHF,), jnp.int32),
        pltpu.VMEM((NHF,), jnp.int32),
        pltpu.VMEM((NHF, C), jnp.float32),
        pltpu.VMEM((NHF, C), jnp.float32),
        pltpu.VMEM((C, NCH), jnp.float32),
        pltpu.VMEM((C, NCH), jnp.float32),
        pltpu.VMEM((NPOINT,), jnp.int32),
        pltpu.VMEM((K,), jnp.float32),
        pltpu.VMEM((NPOINT,), jnp.float32),
        pltpu.SemaphoreType.DMA,
        pltpu.SemaphoreType.DMA,
        pltpu.SemaphoreType.DMA,
        pltpu.SemaphoreType.DMA,
    ],
)
def _sc_gather(xyzp_hbm, featr_hbm, idx_hbm, oxyz_hbm, ofeat_hbm,
               idxq_v, gidx0_v, gidx1_v, src0_v, src1_v, dst0_v, dst1_v,
               idxf_v, xrow_v, oxrow_v, sem_i0, sem_i1, sem_o0, sem_o1):
    cid = lax.axis_index("c")
    sid = lax.axis_index("s")
    wid = sid * 2 + cid
    b = wid // WPB
    q = wid % WPB

    iota = lax.iota(jnp.int32, 16)
    gidx = [gidx0_v, gidx1_v]
    src = [src0_v, src1_v]
    dst = [dst0_v, dst1_v]
    sem_in = [sem_i0, sem_i1]
    sem_out = [sem_o0, sem_o1]
    rbase = b * K
    perm = [(iota + d) & 15 for d in range(16)]

    def fill_gidx(j, h, gbuf):
        for u in range(NHF // 16):
            iv = idxq_v[pl.ds(j * NCH + h * NHF + u * 16, 16)]
            gbuf[pl.ds(u * 16, 16)] = iv + rbase

    pltpu.sync_copy(idx_hbm.at[b, pl.ds(q * PPW, PPW)], idxq_v)
    for h in range(2):
        fill_gidx(0, h, gidx[h])
        pltpu.async_copy(featr_hbm.at[gidx[h]], src[h], sem_in[h])

    @pl.when(wid < 24)
    def _():
        xd = wid % 3
        xb = wid // 3
        pltpu.sync_copy(idx_hbm.at[xb], idxf_v)
        pltpu.sync_copy(xyzp_hbm.at[xd, xb], xrow_v)

        @plsc.parallel_loop(0, NPOINT // 16, unroll=4)
        def xyz_body(k):
            iv = idxf_v[pl.ds(k * 16, 16)]
            oxrow_v[pl.ds(k * 16, 16)] = plsc.load_gather(xrow_v, [iv])

        pltpu.sync_copy(oxrow_v, oxyz_hbm.at[xd, xb])

    def transpose_half(sb, db, h):
        @plsc.parallel_loop(0, TILES, unroll=2)
        def tile_body(z):
            tn = z // (C // 16)
            tc = z % (C // 16)
            rows = tn * 16 + iota
            c0 = tc * 16
            for d in range(16):
                cols = c0 + perm[d]
                v = plsc.load_gather(sb, [rows, cols])
                plsc.store_scatter(db, [cols, rows + h * NHF], v)

    def step_body(t, _):
        for i in range(2):
            j = t * 2 + i
            db = dst[i]

            @pl.when(t > 0)
            def _():
                pltpu.make_async_copy(
                    db, ofeat_hbm.at[b, :, pl.ds(0, NCH)], sem_out[i]).wait()

            for h in range(2):
                pltpu.make_async_copy(
                    featr_hbm.at[gidx[h]], src[h], sem_in[h]).wait()
                transpose_half(src[h], db, h)

                @pl.when(j < NCHUNK - 1)
                def _():
                    fill_gidx(j + 1, h, gidx[h])
                    pltpu.async_copy(featr_hbm.at[gidx[h]], src[h], sem_in[h])

            n0 = q * PPW + j * NCH
            pltpu.async_copy(db, ofeat_hbm.at[b, :, pl.ds(n0, NCH)], sem_out[i])

        return 0

    lax.fori_loop(0, NCHUNK // 2, step_body, 0)

    for i in range(2):
        pltpu.make_async_copy(
            dst[i], ofeat_hbm.at[b, :, pl.ds(0, NCH)], sem_out[i]).wait()


def kernel(xyz, features, sample_inds):
    idx32 = sample_inds.astype(jnp.int32)
    xyzp = xyz.transpose(2, 0, 1)
    featr = features.transpose(0, 2, 1).reshape(B * K, C)
    oxyz, ofeat = _sc_gather(xyzp, featr, idx32)
    new_xyz = oxyz.transpose(1, 2, 0)
    return (new_xyz, ofeat, sample_inds)
